# baseline (speedup 1.0000x reference)
import dataclasses
import functools

import jax
import jax.numpy as jnp
from jax import lax
from jax.experimental import pallas as pl
from jax.experimental.pallas import tpu as pltpu
from jax.experimental.pallas import tpu_sc as plsc

N_NODES = 10000
N_PAD = 10240
D_PAD = 128
N_EDGES = 320000
NC = 2
NS = 16
NW = NC * NS
CHUNK = 128
CPW = 80
E_PAD = CPW * NW * CHUNK
DOT_CHUNK = 128
DOT_CPW = E_PAD // (NW * DOT_CHUNK)


def _vector_mesh():
    return plsc.VectorSubcoreMesh(core_axis_name="c", subcore_axis_name="s")


def _layout_workaround_params():
    cp = pltpu.CompilerParams()
    if "needs_layout_passes" in pltpu.CompilerParams.__dataclass_fields__:
        cp = dataclasses.replace(cp, needs_layout_passes=False)
    return cp


def _sc_degree(dst_r):
    @functools.partial(
        pl.kernel,
        out_type=jax.ShapeDtypeStruct((NC, N_PAD), jnp.float32),
        mesh=_vector_mesh(),
        scratch_types=[
            pltpu.VMEM_SHARED((N_PAD,), jnp.float32),
            pltpu.VMEM((CPW, CHUNK), jnp.int32),
            pltpu.VMEM((CHUNK,), jnp.float32),
        ],
    )
    def k(dst_hbm, zeros_hbm, ones_hbm, out_hbm, acc_sh, idx_v, ones_v):
        cid = lax.axis_index("c")
        sid = lax.axis_index("s")
        wid = sid * NC + cid
        rpt = N_PAD // NS
        r0 = sid * rpt
        pltpu.sync_copy(ones_hbm, ones_v)
        pltpu.sync_copy(dst_hbm.at[pl.ds(wid * CPW, CPW)], idx_v)
        pltpu.sync_copy(zeros_hbm.at[pl.ds(r0, rpt)], acc_sh.at[pl.ds(r0, rpt)])
        plsc.subcore_barrier()

        @pl.loop(0, CPW)
        def _(t):
            pltpu.sync_copy(ones_v, acc_sh.at[idx_v.at[t]], add=True)

        plsc.subcore_barrier()
        pltpu.sync_copy(acc_sh.at[pl.ds(r0, rpt)],
                        out_hbm.at[cid].at[pl.ds(r0, rpt)])

    return k(dst_r, jnp.zeros((N_PAD,), jnp.float32),
             jnp.ones((CHUNK,), jnp.float32))


def _sc_aggregate(h, src_p, dst_p, zeros2d):
    n, d = h.shape
    assert d == D_PAD

    @functools.partial(
        pl.kernel,
        out_type=jax.ShapeDtypeStruct((NC, N_PAD, d), jnp.float32),
        mesh=_vector_mesh(),
        scratch_types=[
            pltpu.VMEM_SHARED((N_PAD, d), jnp.float32),
            pltpu.VMEM((CHUNK,), jnp.int32),
            pltpu.VMEM((CHUNK,), jnp.int32),
            pltpu.VMEM((CHUNK, d), jnp.float32),
        ],
    )
    def k(h_hbm, src_hbm, dst_hbm, z_hbm, out_hbm, acc_sh, src_v, dst_v,
          rows_v):
        cid = lax.axis_index("c")
        sid = lax.axis_index("s")
        wid = sid * NC + cid
        rpt = N_PAD // NS
        row0 = sid * rpt
        pltpu.sync_copy(z_hbm.at[pl.ds(row0, rpt)], acc_sh.at[pl.ds(row0, rpt)])
        plsc.subcore_barrier()

        @pl.loop(0, CPW)
        def _(t):
            base = (wid + t * NW) * CHUNK
            pltpu.sync_copy(src_hbm.at[pl.ds(base, CHUNK)], src_v)
            pltpu.sync_copy(dst_hbm.at[pl.ds(base, CHUNK)], dst_v)
            pltpu.sync_copy(h_hbm.at[src_v], rows_v)
            pltpu.sync_copy(rows_v, acc_sh.at[dst_v], add=True)

        plsc.subcore_barrier()
        pltpu.sync_copy(acc_sh.at[pl.ds(row0, rpt)],
                        out_hbm.at[cid].at[pl.ds(row0, rpt)])

    return k(h, src_p, dst_p, zeros2d)


def _sc_edge_dot(z, src_p, dst_p, d_real):
    n, d = z.shape
    assert d == D_PAD

    @functools.partial(
        pl.kernel,
        out_type=jax.ShapeDtypeStruct((E_PAD,), jnp.float32),
        mesh=_vector_mesh(),
        compiler_params=_layout_workaround_params(),
        scratch_types=[
            pltpu.VMEM((CHUNK,), jnp.int32),
            pltpu.VMEM((CHUNK,), jnp.int32),
            pltpu.VMEM((CHUNK, d), jnp.float32),
            pltpu.VMEM((CHUNK, d), jnp.float32),
            pltpu.VMEM((CHUNK,), jnp.float32),
        ],
    )
    def k(z_hbm, src_hbm, dst_hbm, out_hbm, src_v, dst_v, a_v, b_v, o_v):
        cid = lax.axis_index("c")
        sid = lax.axis_index("s")
        wid = sid * NC + cid

        @pl.loop(0, CPW)
        def _(t):
            base = (wid + t * NW) * CHUNK
            pltpu.sync_copy(src_hbm.at[pl.ds(base, CHUNK)], src_v)
            pltpu.sync_copy(dst_hbm.at[pl.ds(base, CHUNK)], dst_v)
            pltpu.sync_copy(z_hbm.at[src_v], a_v)
            pltpu.sync_copy(z_hbm.at[dst_v], b_v)

            @pl.loop(0, CHUNK, step=16)
            def _(e):
                rows = e + lax.iota(jnp.int32, 16)
                acc = jnp.zeros((16,), jnp.float32)
                for col in range(d_real):
                    cols = jnp.full((16,), col, jnp.int32)
                    va = plsc.load_gather(a_v, [rows, cols])
                    vb = plsc.load_gather(b_v, [rows, cols])
                    acc = acc + va * vb
                o_v[pl.ds(e, 16)] = acc

            pltpu.sync_copy(o_v, out_hbm.at[pl.ds(base, CHUNK)])

    return k(z, src_p, dst_p)


def _tc_matmul(x, w1):
    n = x.shape[0]
    h = w1.shape[1]

    def body(x_ref, w_ref, h0_ref):
        h0_ref[...] = jnp.dot(x_ref[...], w_ref[...],
                              preferred_element_type=jnp.float32)

    return pl.pallas_call(
        body,
        out_shape=jax.ShapeDtypeStruct((n, h), jnp.float32),
    )(x, w1)


def _tc_stage1(h0, deg_parts):
    n, h = h0.shape

    def body(h0_ref, deg_ref, h0p_ref, dis_ref):
        deg = deg_ref[0, :n] + deg_ref[1, :n] + 1.0
        dis = lax.rsqrt(deg)
        h0p_ref[:, :h] = h0_ref[...] * dis[:, None]
        h0p_ref[:, h:] = jnp.zeros((n, D_PAD - h), jnp.float32)
        dis_ref[...] = dis[:, None]

    return pl.pallas_call(
        body,
        out_shape=(jax.ShapeDtypeStruct((n, D_PAD), jnp.float32),
                   jax.ShapeDtypeStruct((n, 1), jnp.float32)),
    )(h0, deg_parts)


def _tc_stage2(acc_parts, h0p, dis, b1):
    n = h0p.shape[0]
    h = b1.shape[1]

    def body(acc_ref, h0p_ref, dis_ref, b1_ref, h1p_ref):
        s = acc_ref[0, :n, :h] + acc_ref[1, :n, :h] + h0p_ref[:, :h]
        h1 = jnp.maximum(s * dis_ref[...] + b1_ref[...], 0.0)
        h1p_ref[:, :h] = h1 * dis_ref[...]
        h1p_ref[:, h:] = jnp.zeros((n, D_PAD - h), jnp.float32)

    return pl.pallas_call(
        body,
        out_shape=jax.ShapeDtypeStruct((n, D_PAD), jnp.float32),
    )(acc_parts, h0p, dis, b1)


def _tc_stage3(acc_parts, h1p, dis, w_mu, b_mu, w_ls, b_ls):
    n = h1p.shape[0]
    h = w_mu.shape[0]
    o = w_mu.shape[1]

    def body(acc_ref, h1p_ref, dis_ref, wm_ref, bm_ref, wl_ref, bl_ref,
             mu_ref, ls_ref, mup_ref):
        g = (acc_ref[0, :n, :h] + acc_ref[1, :n, :h] + h1p_ref[:, :h]) \
            * dis_ref[...]
        mu = jnp.dot(g, wm_ref[...],
                     preferred_element_type=jnp.float32) + bm_ref[...]
        mu_ref[...] = mu
        ls_ref[...] = jnp.dot(g, wl_ref[...],
                              preferred_element_type=jnp.float32) + bl_ref[...]
        mup_ref[:n, :o] = mu
        mup_ref[:n, o:] = jnp.zeros((n, D_PAD - o), jnp.float32)
        mup_ref[n:, :] = jnp.zeros((N_PAD - n, D_PAD), jnp.float32)

    return pl.pallas_call(
        body,
        out_shape=(jax.ShapeDtypeStruct((n, o), jnp.float32),
                   jax.ShapeDtypeStruct((n, o), jnp.float32),
                   jax.ShapeDtypeStruct((N_PAD, D_PAD), jnp.float32)),
    )(acc_parts, h1p, dis, w_mu, b_mu, w_ls, b_ls)


def kernel(x, edge_index, W1, b1, W_mu, b_mu, W_ls, b_ls):
    src = edge_index[0].astype(jnp.int32)
    dst = edge_index[1].astype(jnp.int32)
    pad = E_PAD - N_EDGES
    src_p = jnp.concatenate([src, jnp.zeros((pad,), jnp.int32)])
    dst_p = jnp.concatenate([dst, jnp.full((pad,), N_PAD - 1, jnp.int32)])
    dst_r = dst_p.reshape(CPW * NW, CHUNK)
    zeros2d = jnp.zeros((N_PAD, D_PAD), jnp.float32)

    deg_parts = _sc_degree(dst_r)
    h0 = _tc_matmul(x, W1)
    h0p, dis = _tc_stage1(h0, deg_parts)
    acc1 = _sc_aggregate(h0p, src_p, dst_p, zeros2d)
    h1p = _tc_stage2(acc1, h0p, dis, b1.reshape(1, -1))
    acc2 = _sc_aggregate(h1p, src_p, dst_p, zeros2d)
    mu, logstd, mu_pad = _tc_stage3(acc2, h1p, dis, W_mu, b_mu.reshape(1, -1),
                                    W_ls, b_ls.reshape(1, -1))
    adj_full = _sc_edge_dot(mu_pad, src_p, dst_p, W_mu.shape[1])
    return adj_full[:N_EDGES], mu, logstd


# speedup vs baseline: 1.7468x; 1.7468x over previous
import dataclasses
import functools

import jax
import jax.numpy as jnp
from jax import lax
from jax.experimental import pallas as pl
from jax.experimental.pallas import tpu as pltpu
from jax.experimental.pallas import tpu_sc as plsc

N_NODES = 10000
N_PAD = 10240
D_PAD = 128
N_EDGES = 320000
NC = 2
NS = 16
NW = NC * NS
CHUNK = 128
CPW = 80
E_PAD = CPW * NW * CHUNK
DOT_CHUNK = 128
DOT_CPW = E_PAD // (NW * DOT_CHUNK)


def _vector_mesh():
    return plsc.VectorSubcoreMesh(core_axis_name="c", subcore_axis_name="s")


def _layout_workaround_params():
    cp = pltpu.CompilerParams()
    if "needs_layout_passes" in pltpu.CompilerParams.__dataclass_fields__:
        cp = dataclasses.replace(cp, needs_layout_passes=False)
    return cp


def _sc_degree(dst_r):
    @functools.partial(
        pl.kernel,
        out_type=jax.ShapeDtypeStruct((NC, N_PAD), jnp.float32),
        mesh=_vector_mesh(),
        scratch_types=[
            pltpu.VMEM_SHARED((N_PAD,), jnp.float32),
            pltpu.VMEM((CPW, CHUNK), jnp.int32),
            pltpu.VMEM((CHUNK,), jnp.float32),
        ],
    )
    def k(dst_hbm, zeros_hbm, ones_hbm, out_hbm, acc_sh, idx_v, ones_v):
        cid = lax.axis_index("c")
        sid = lax.axis_index("s")
        wid = sid * NC + cid
        rpt = N_PAD // NS
        r0 = sid * rpt
        pltpu.sync_copy(ones_hbm, ones_v)
        pltpu.sync_copy(dst_hbm.at[pl.ds(wid * CPW, CPW)], idx_v)
        pltpu.sync_copy(zeros_hbm.at[pl.ds(r0, rpt)], acc_sh.at[pl.ds(r0, rpt)])
        plsc.subcore_barrier()

        @pl.loop(0, CPW)
        def _(t):
            pltpu.sync_copy(ones_v, acc_sh.at[idx_v.at[t]], add=True)

        plsc.subcore_barrier()
        pltpu.sync_copy(acc_sh.at[pl.ds(r0, rpt)],
                        out_hbm.at[cid].at[pl.ds(r0, rpt)])

    return k(dst_r, jnp.zeros((N_PAD,), jnp.float32),
             jnp.ones((CHUNK,), jnp.float32))


def _sc_aggregate(h, src_p, dst_p, zeros2d):
    n, d = h.shape
    assert d == D_PAD

    @functools.partial(
        pl.kernel,
        out_type=jax.ShapeDtypeStruct((NC, N_PAD, d), jnp.float32),
        mesh=_vector_mesh(),
        scratch_types=[
            pltpu.VMEM_SHARED((N_PAD, d), jnp.float32),
            pltpu.VMEM((CHUNK,), jnp.int32),
            pltpu.VMEM((CHUNK,), jnp.int32),
            pltpu.VMEM((CHUNK, d), jnp.float32),
        ],
    )
    def k(h_hbm, src_hbm, dst_hbm, z_hbm, out_hbm, acc_sh, src_v, dst_v,
          rows_v):
        cid = lax.axis_index("c")
        sid = lax.axis_index("s")
        wid = sid * NC + cid
        rpt = N_PAD // NS
        row0 = sid * rpt
        pltpu.sync_copy(z_hbm.at[pl.ds(row0, rpt)], acc_sh.at[pl.ds(row0, rpt)])
        plsc.subcore_barrier()

        @pl.loop(0, CPW)
        def _(t):
            base = (wid + t * NW) * CHUNK
            pltpu.sync_copy(src_hbm.at[pl.ds(base, CHUNK)], src_v)
            pltpu.sync_copy(dst_hbm.at[pl.ds(base, CHUNK)], dst_v)
            pltpu.sync_copy(h_hbm.at[src_v], rows_v)
            pltpu.sync_copy(rows_v, acc_sh.at[dst_v], add=True)

        plsc.subcore_barrier()
        pltpu.sync_copy(acc_sh.at[pl.ds(row0, rpt)],
                        out_hbm.at[cid].at[pl.ds(row0, rpt)])

    return k(h, src_p, dst_p, zeros2d)


def _sc_edge_dot(z, src_p, dst_p, d_real):
    n, d = z.shape
    assert d == D_PAD

    @functools.partial(
        pl.kernel,
        out_type=jax.ShapeDtypeStruct((E_PAD,), jnp.float32),
        mesh=_vector_mesh(),
        compiler_params=_layout_workaround_params(),
        scratch_types=[
            pltpu.VMEM((CHUNK,), jnp.int32),
            pltpu.VMEM((CHUNK,), jnp.int32),
            pltpu.VMEM((CHUNK, d), jnp.float32),
            pltpu.VMEM((CHUNK, d), jnp.float32),
            pltpu.VMEM((CHUNK,), jnp.float32),
        ],
    )
    def k(z_hbm, src_hbm, dst_hbm, out_hbm, src_v, dst_v, a_v, b_v, o_v):
        cid = lax.axis_index("c")
        sid = lax.axis_index("s")
        wid = sid * NC + cid

        @pl.loop(0, CPW)
        def _(t):
            base = (wid + t * NW) * CHUNK
            pltpu.sync_copy(src_hbm.at[pl.ds(base, CHUNK)], src_v)
            pltpu.sync_copy(dst_hbm.at[pl.ds(base, CHUNK)], dst_v)
            pltpu.sync_copy(z_hbm.at[src_v], a_v)
            pltpu.sync_copy(z_hbm.at[dst_v], b_v)

            @pl.loop(0, CHUNK, step=16)
            def _(e):
                rows = e + lax.iota(jnp.int32, 16)
                acc = jnp.zeros((16,), jnp.float32)
                for col in range(d_real):
                    cols = jnp.full((16,), col, jnp.int32)
                    va = plsc.load_gather(a_v, [rows, cols])
                    vb = plsc.load_gather(b_v, [rows, cols])
                    acc = acc + va * vb
                o_v[pl.ds(e, 16)] = acc

            pltpu.sync_copy(o_v, out_hbm.at[pl.ds(base, CHUNK)])

    return k(z, src_p, dst_p)


def _tc_matmul(x, w1):
    n = x.shape[0]
    h = w1.shape[1]

    def body(x_ref, w_ref, h0_ref):
        h0_ref[...] = jnp.dot(x_ref[...], w_ref[...],
                              preferred_element_type=jnp.float32)

    return pl.pallas_call(
        body,
        out_shape=jax.ShapeDtypeStruct((n, h), jnp.float32),
    )(x, w1)


def _tc_stage1(h0, deg_parts):
    n, h = h0.shape

    def body(h0_ref, deg_ref, h0p_ref, dis_ref):
        deg = deg_ref[0, :n] + deg_ref[1, :n] + 1.0
        dis = lax.rsqrt(deg)
        h0p_ref[:, :h] = h0_ref[...] * dis[:, None]
        h0p_ref[:, h:] = jnp.zeros((n, D_PAD - h), jnp.float32)
        dis_ref[...] = dis[:, None]

    return pl.pallas_call(
        body,
        out_shape=(jax.ShapeDtypeStruct((n, D_PAD), jnp.float32),
                   jax.ShapeDtypeStruct((n, 1), jnp.float32)),
    )(h0, deg_parts)


def _tc_stage2(acc_parts, h0p, dis, b1):
    n = h0p.shape[0]
    h = b1.shape[1]

    def body(acc_ref, h0p_ref, dis_ref, b1_ref, h1p_ref):
        s = acc_ref[0, :n, :h] + acc_ref[1, :n, :h] + h0p_ref[:, :h]
        h1 = jnp.maximum(s * dis_ref[...] + b1_ref[...], 0.0)
        h1p_ref[:, :h] = h1 * dis_ref[...]
        h1p_ref[:, h:] = jnp.zeros((n, D_PAD - h), jnp.float32)

    return pl.pallas_call(
        body,
        out_shape=jax.ShapeDtypeStruct((n, D_PAD), jnp.float32),
    )(acc_parts, h0p, dis, b1)


def _tc_stage3(acc_parts, h1p, dis, w_mu, b_mu, w_ls, b_ls):
    n = h1p.shape[0]
    h = w_mu.shape[0]
    o = w_mu.shape[1]

    def body(acc_ref, h1p_ref, dis_ref, wm_ref, bm_ref, wl_ref, bl_ref,
             mu_ref, ls_ref, mup_ref):
        g = (acc_ref[0, :n, :h] + acc_ref[1, :n, :h] + h1p_ref[:, :h]) \
            * dis_ref[...]
        mu = jnp.dot(g, wm_ref[...],
                     preferred_element_type=jnp.float32) + bm_ref[...]
        mu_ref[...] = mu
        ls_ref[...] = jnp.dot(g, wl_ref[...],
                              preferred_element_type=jnp.float32) + bl_ref[...]
        mup_ref[:n, :o] = mu
        mup_ref[:n, o:] = jnp.zeros((n, D_PAD - o), jnp.float32)
        mup_ref[n:, :] = jnp.zeros((N_PAD - n, D_PAD), jnp.float32)

    return pl.pallas_call(
        body,
        out_shape=(jax.ShapeDtypeStruct((n, o), jnp.float32),
                   jax.ShapeDtypeStruct((n, o), jnp.float32),
                   jax.ShapeDtypeStruct((N_PAD, D_PAD), jnp.float32)),
    )(acc_parts, h1p, dis, w_mu, b_mu, w_ls, b_ls)


def kernel(x, edge_index, W1, b1, W_mu, b_mu, W_ls, b_ls):
    src = edge_index[0].astype(jnp.int32)
    dst = edge_index[1].astype(jnp.int32)
    pad = E_PAD - N_EDGES
    pad_src = jnp.arange(pad, dtype=jnp.int32) % N_NODES
    pad_dst = N_NODES + jnp.arange(pad, dtype=jnp.int32) % (N_PAD - N_NODES)
    src_p = jnp.concatenate([src, pad_src])
    dst_p = jnp.concatenate([dst, pad_dst])
    dst_r = dst_p.reshape(CPW * NW, CHUNK)
    zeros2d = jnp.zeros((N_PAD, D_PAD), jnp.float32)

    deg_parts = _sc_degree(dst_r)
    h0 = _tc_matmul(x, W1)
    h0p, dis = _tc_stage1(h0, deg_parts)
    acc1 = _sc_aggregate(h0p, src_p, dst_p, zeros2d)
    h1p = _tc_stage2(acc1, h0p, dis, b1.reshape(1, -1))
    acc2 = _sc_aggregate(h1p, src_p, dst_p, zeros2d)
    mu, logstd, mu_pad = _tc_stage3(acc2, h1p, dis, W_mu, b_mu.reshape(1, -1),
                                    W_ls, b_ls.reshape(1, -1))
    adj_full = _sc_edge_dot(mu_pad, src_p, dst_p, W_mu.shape[1])
    return adj_full[:N_EDGES], mu, logstd


# speedup vs baseline: 3.0899x; 1.7689x over previous
import dataclasses
import functools

import jax
import jax.numpy as jnp
from jax import lax
from jax.experimental import pallas as pl
from jax.experimental.pallas import tpu as pltpu
from jax.experimental.pallas import tpu_sc as plsc

N_NODES = 10000
N_PAD = 10240
D_PAD = 128
N_EDGES = 320000
NC = 2
NS = 16
NW = NC * NS
CHUNK = 128
CPW = 80
E_PAD = CPW * NW * CHUNK
DOT_CHUNK = 128
DOT_CPW = E_PAD // (NW * DOT_CHUNK)


def _vector_mesh():
    return plsc.VectorSubcoreMesh(core_axis_name="c", subcore_axis_name="s")


def _layout_workaround_params():
    cp = pltpu.CompilerParams()
    if "needs_layout_passes" in pltpu.CompilerParams.__dataclass_fields__:
        cp = dataclasses.replace(cp, needs_layout_passes=False)
    return cp


def _sc_degree(dst_r):
    @functools.partial(
        pl.kernel,
        out_type=jax.ShapeDtypeStruct((NC, N_PAD), jnp.float32),
        mesh=_vector_mesh(),
        scratch_types=[
            pltpu.VMEM_SHARED((N_PAD,), jnp.float32),
            pltpu.VMEM((CPW, CHUNK), jnp.int32),
            pltpu.VMEM((CHUNK,), jnp.float32),
        ],
    )
    def k(dst_hbm, zeros_hbm, ones_hbm, out_hbm, acc_sh, idx_v, ones_v):
        cid = lax.axis_index("c")
        sid = lax.axis_index("s")
        wid = sid * NC + cid
        rpt = N_PAD // NS
        r0 = sid * rpt
        pltpu.sync_copy(ones_hbm, ones_v)
        pltpu.sync_copy(dst_hbm.at[pl.ds(wid * CPW, CPW)], idx_v)
        pltpu.sync_copy(zeros_hbm.at[pl.ds(r0, rpt)], acc_sh.at[pl.ds(r0, rpt)])
        plsc.subcore_barrier()

        @pl.loop(0, CPW)
        def _(t):
            pltpu.sync_copy(ones_v, acc_sh.at[idx_v.at[t]], add=True)

        plsc.subcore_barrier()
        pltpu.sync_copy(acc_sh.at[pl.ds(r0, rpt)],
                        out_hbm.at[cid].at[pl.ds(r0, rpt)])

    return k(dst_r, jnp.zeros((N_PAD,), jnp.float32),
             jnp.ones((CHUNK,), jnp.float32))


def _sc_aggregate(h, src_r, dst_r, zeros2d):
    n, d = h.shape
    assert d == D_PAD
    hcpw = CPW // 2

    @functools.partial(
        pl.kernel,
        out_type=jax.ShapeDtypeStruct((NC, N_PAD, d), jnp.float32),
        mesh=_vector_mesh(),
        scratch_types=[
            pltpu.VMEM_SHARED((N_PAD, d), jnp.float32),
            pltpu.VMEM((hcpw, CHUNK), jnp.int32),
            pltpu.VMEM((hcpw, CHUNK), jnp.int32),
            pltpu.VMEM((CHUNK, d), jnp.float32),
            pltpu.VMEM((CHUNK, d), jnp.float32),
            pltpu.SemaphoreType.DMA,
            pltpu.SemaphoreType.DMA,
        ],
    )
    def k(h_hbm, src_hbm, dst_hbm, z_hbm, out_hbm, acc_sh,
          sidx, didx, rb0, rb1, sg0, sg1):
        cid = lax.axis_index("c")
        sid = lax.axis_index("s")
        wid = sid * NC + cid
        rpt = N_PAD // NS
        row0 = sid * rpt
        rbuf = [rb0, rb1]
        sem_g = [sg0, sg1]
        pltpu.sync_copy(z_hbm.at[pl.ds(row0, rpt)], acc_sh.at[pl.ds(row0, rpt)])
        plsc.subcore_barrier()

        def gather(b, t):
            return pltpu.make_async_copy(h_hbm.at[sidx.at[t]], rbuf[b],
                                         sem_g[b])

        for ph in range(2):
            base = wid * CPW + ph * hcpw
            pltpu.sync_copy(src_hbm.at[pl.ds(base, hcpw)], sidx)
            pltpu.sync_copy(dst_hbm.at[pl.ds(base, hcpw)], didx)
            gather(0, 0).start()

            @pl.loop(0, hcpw, step=2)
            def _(t):
                for b in range(2):
                    tt = t + b
                    gather(b, tt).wait()

                    @pl.when(tt + 1 < hcpw)
                    def _():
                        gather(1 - b, tt + 1).start()

                    pltpu.sync_copy(rbuf[b], acc_sh.at[didx.at[tt]], add=True)

        plsc.subcore_barrier()
        pltpu.sync_copy(acc_sh.at[pl.ds(row0, rpt)],
                        out_hbm.at[cid].at[pl.ds(row0, rpt)])

    return k(h, src_r, dst_r, zeros2d)


def _sc_edge_dot(z, src_r, dst_r, d_real):
    n, d = z.shape
    assert d == D_PAD

    @functools.partial(
        pl.kernel,
        out_type=jax.ShapeDtypeStruct((CPW * NW, CHUNK), jnp.float32),
        mesh=_vector_mesh(),
        compiler_params=_layout_workaround_params(),
        scratch_types=[
            pltpu.VMEM((CPW, CHUNK), jnp.int32),
            pltpu.VMEM((CPW, CHUNK), jnp.int32),
            pltpu.VMEM((CHUNK, d), jnp.float32),
            pltpu.VMEM((CHUNK, d), jnp.float32),
            pltpu.VMEM((CHUNK, d), jnp.float32),
            pltpu.VMEM((CHUNK, d), jnp.float32),
            pltpu.VMEM((CPW, CHUNK), jnp.float32),
            pltpu.SemaphoreType.DMA,
            pltpu.SemaphoreType.DMA,
        ],
    )
    def k(z_hbm, src_hbm, dst_hbm, out_hbm,
          sidx, didx, a0, a1, b0, b1, o_all, sg0, sg1):
        cid = lax.axis_index("c")
        sid = lax.axis_index("s")
        wid = sid * NC + cid
        abuf = [a0, a1]
        bbuf = [b0, b1]
        sem_g = [sg0, sg1]
        pltpu.sync_copy(src_hbm.at[pl.ds(wid * CPW, CPW)], sidx)
        pltpu.sync_copy(dst_hbm.at[pl.ds(wid * CPW, CPW)], didx)

        def gathers(b, t):
            return (pltpu.make_async_copy(z_hbm.at[sidx.at[t]], abuf[b],
                                          sem_g[b]),
                    pltpu.make_async_copy(z_hbm.at[didx.at[t]], bbuf[b],
                                          sem_g[b]))

        for c in gathers(0, 0):
            c.start()

        @pl.loop(0, CPW, step=2)
        def _(t):
            for b in range(2):
                tt = t + b
                for c in gathers(b, tt):
                    c.wait()

                @pl.when(tt + 1 < CPW)
                def _():
                    for c in gathers(1 - b, tt + 1):
                        c.start()

                @pl.loop(0, CHUNK, step=16)
                def _(e):
                    rows = e + lax.iota(jnp.int32, 16)
                    acc = jnp.zeros((16,), jnp.float32)
                    for col in range(d_real):
                        cols = jnp.full((16,), col, jnp.int32)
                        va = plsc.load_gather(abuf[b], [rows, cols])
                        vb = plsc.load_gather(bbuf[b], [rows, cols])
                        acc = acc + va * vb
                    o_all[tt, pl.ds(e, 16)] = acc

        pltpu.sync_copy(o_all, out_hbm.at[pl.ds(wid * CPW, CPW)])

    return k(z, src_r, dst_r)


def _tc_matmul(x, w1):
    n = x.shape[0]
    h = w1.shape[1]

    def body(x_ref, w_ref, h0_ref):
        h0_ref[...] = jnp.dot(x_ref[...], w_ref[...],
                              preferred_element_type=jnp.float32)

    return pl.pallas_call(
        body,
        out_shape=jax.ShapeDtypeStruct((n, h), jnp.float32),
    )(x, w1)


def _tc_stage1(h0, deg_parts):
    n, h = h0.shape

    def body(h0_ref, deg_ref, h0p_ref, dis_ref):
        deg = deg_ref[0, :n] + deg_ref[1, :n] + 1.0
        dis = lax.rsqrt(deg)
        h0p_ref[:, :h] = h0_ref[...] * dis[:, None]
        h0p_ref[:, h:] = jnp.zeros((n, D_PAD - h), jnp.float32)
        dis_ref[...] = dis[:, None]

    return pl.pallas_call(
        body,
        out_shape=(jax.ShapeDtypeStruct((n, D_PAD), jnp.float32),
                   jax.ShapeDtypeStruct((n, 1), jnp.float32)),
    )(h0, deg_parts)


def _tc_stage2(acc_parts, h0p, dis, b1):
    n = h0p.shape[0]
    h = b1.shape[1]

    def body(acc_ref, h0p_ref, dis_ref, b1_ref, h1p_ref):
        s = acc_ref[0, :n, :h] + acc_ref[1, :n, :h] + h0p_ref[:, :h]
        h1 = jnp.maximum(s * dis_ref[...] + b1_ref[...], 0.0)
        h1p_ref[:, :h] = h1 * dis_ref[...]
        h1p_ref[:, h:] = jnp.zeros((n, D_PAD - h), jnp.float32)

    return pl.pallas_call(
        body,
        out_shape=jax.ShapeDtypeStruct((n, D_PAD), jnp.float32),
    )(acc_parts, h0p, dis, b1)


def _tc_stage3(acc_parts, h1p, dis, w_mu, b_mu, w_ls, b_ls):
    n = h1p.shape[0]
    h = w_mu.shape[0]
    o = w_mu.shape[1]

    def body(acc_ref, h1p_ref, dis_ref, wm_ref, bm_ref, wl_ref, bl_ref,
             mu_ref, ls_ref, mup_ref):
        g = (acc_ref[0, :n, :h] + acc_ref[1, :n, :h] + h1p_ref[:, :h]) \
            * dis_ref[...]
        mu = jnp.dot(g, wm_ref[...],
                     preferred_element_type=jnp.float32) + bm_ref[...]
        mu_ref[...] = mu
        ls_ref[...] = jnp.dot(g, wl_ref[...],
                              preferred_element_type=jnp.float32) + bl_ref[...]
        mup_ref[:n, :o] = mu
        mup_ref[:n, o:] = jnp.zeros((n, D_PAD - o), jnp.float32)
        mup_ref[n:, :] = jnp.zeros((N_PAD - n, D_PAD), jnp.float32)

    return pl.pallas_call(
        body,
        out_shape=(jax.ShapeDtypeStruct((n, o), jnp.float32),
                   jax.ShapeDtypeStruct((n, o), jnp.float32),
                   jax.ShapeDtypeStruct((N_PAD, D_PAD), jnp.float32)),
    )(acc_parts, h1p, dis, w_mu, b_mu, w_ls, b_ls)


def kernel(x, edge_index, W1, b1, W_mu, b_mu, W_ls, b_ls):
    src = edge_index[0].astype(jnp.int32)
    dst = edge_index[1].astype(jnp.int32)
    pad = E_PAD - N_EDGES
    pad_src = jnp.arange(pad, dtype=jnp.int32) % N_NODES
    pad_dst = N_NODES + jnp.arange(pad, dtype=jnp.int32) % (N_PAD - N_NODES)
    src_r = jnp.concatenate([src, pad_src]).reshape(CPW * NW, CHUNK)
    dst_r = jnp.concatenate([dst, pad_dst]).reshape(CPW * NW, CHUNK)
    zeros2d = jnp.zeros((N_PAD, D_PAD), jnp.float32)

    deg_parts = _sc_degree(dst_r)
    h0 = _tc_matmul(x, W1)
    h0p, dis = _tc_stage1(h0, deg_parts)
    acc1 = _sc_aggregate(h0p, src_r, dst_r, zeros2d)
    h1p = _tc_stage2(acc1, h0p, dis, b1.reshape(1, -1))
    acc2 = _sc_aggregate(h1p, src_r, dst_r, zeros2d)
    mu, logstd, mu_pad = _tc_stage3(acc2, h1p, dis, W_mu, b_mu.reshape(1, -1),
                                    W_ls, b_ls.reshape(1, -1))
    adj_full = _sc_edge_dot(mu_pad, src_r, dst_r, W_mu.shape[1]).reshape(-1)
    return adj_full[:N_EDGES], mu, logstd
